# Initial kernel scaffold; baseline (speedup 1.0000x reference)
import jax
import jax.numpy as jnp
from jax.experimental import pallas as pl


def kernel(query, memory, in_proj_w, conv_w, conv_b, x_proj_w, dt_proj_w, dt_proj_b, A_log, D, out_proj_w, ln_w, ln_b):
    raise NotImplementedError("write your pallas kernel here")


# speedup vs baseline: 3.0818x; 3.0818x over previous
import functools

import jax
import jax.numpy as jnp
from jax import lax
from jax.experimental import pallas as pl
from jax.experimental.pallas import tpu as pltpu
from jax.experimental.pallas import tpu_sc as plsc

_D = 128
_M = 1000000
_K = 8
_BLK = 8192
_NBLK = (_M + _BLK - 1) // _BLK
_NCAND = _NBLK * _K
_CHUNKS = (_NCAND + 15) // 16
_CPAD = _CHUNKS * 16
_MINF = float(jnp.finfo(jnp.float32).min)

_D_STATE = 16
_D_CONV = 4
_D_INNER = 256
_DT_RANK = 8


def _sims_topk_kernel(q_ref, mem_ref, vals_ref, idx_ref):
    i = pl.program_id(0)
    q = q_ref[...]
    qn = q / jnp.maximum(jnp.sqrt(jnp.sum(q * q)), 1e-8)

    mem = mem_ref[...]
    dn = (((1,), (1,)), ((), ()))
    s = lax.dot_general(qn, mem, dn,
                        preferred_element_type=jnp.float32)
    ones = jnp.ones((1, _D), dtype=jnp.float32)
    n2 = lax.dot_general(ones, mem * mem, dn,
                         preferred_element_type=jnp.float32)
    sims = s / jnp.maximum(jnp.sqrt(n2), 1e-8)

    gidx = i * _BLK + lax.broadcasted_iota(jnp.int32, (1, _BLK), 1)
    sims = jnp.where(gidx < _M, sims, _MINF)

    big = jnp.int32(2147483647)
    for j in range(_K):
        m = jnp.max(sims)
        loc = jnp.min(jnp.where(sims == m, gidx, big))
        vals_ref[0, 0, j] = m
        idx_ref[0, 0, j] = loc
        sims = jnp.where(gidx == loc, _MINF, sims)


def _run_sims_topk(query, memory):
    q2 = query.reshape(1, _D)
    return pl.pallas_call(
        _sims_topk_kernel,
        grid=(_NBLK,),
        in_specs=[
            pl.BlockSpec((1, _D), lambda i: (0, 0)),
            pl.BlockSpec((_BLK, _D), lambda i: (i, 0)),
        ],
        out_specs=[
            pl.BlockSpec((1, 1, _K), lambda i: (i, 0, 0),
                         memory_space=pltpu.SMEM),
            pl.BlockSpec((1, 1, _K), lambda i: (i, 0, 0),
                         memory_space=pltpu.SMEM),
        ],
        out_shape=[
            jax.ShapeDtypeStruct((_NBLK, 1, _K), jnp.float32),
            jax.ShapeDtypeStruct((_NBLK, 1, _K), jnp.int32),
        ],
    )(q2, memory)


def _sc_merge_gather_body(vals_hbm, idx_hbm, mem_hbm, out_hbm,
                          vals_v, idx_v, topi_v, rows_v, sem):
    cid = lax.axis_index("c")
    sid = lax.axis_index("s")

    @pl.when(jnp.logical_and(cid == 0, sid == 0))
    def _():
        pltpu.sync_copy(vals_hbm, vals_v)
        pltpu.sync_copy(idx_hbm, idx_v)

        def body(c, carry):
            cv, ci = carry
            v = vals_v[pl.ds(c * 16, 16)]
            ii = idx_v[pl.ds(c * 16, 16)]
            sv, si = plsc.sort_key_val(v, ii, descending=True)
            rv = lax.rev(sv, (0,))
            ri = lax.rev(si, (0,))
            keep = cv >= rv
            nv = jnp.where(keep, cv, rv)
            ni = jnp.where(keep, ci, ri)
            mv, mi = plsc.sort_key_val(nv, ni, descending=True)
            return (mv, mi)

        init = (jnp.full((16,), _MINF, jnp.float32),
                jnp.zeros((16,), jnp.int32))
        _, top_i = lax.fori_loop(0, _CHUNKS, body, init)
        topi_v[...] = top_i
        pltpu.async_copy(mem_hbm.at[topi_v], rows_v, sem).wait()
        pltpu.sync_copy(rows_v, out_hbm)


def _run_sc_merge_gather(vals, idx, memory):
    mesh = plsc.VectorSubcoreMesh(core_axis_name="c", subcore_axis_name="s",
                                  num_cores=2, num_subcores=16)
    fn = pl.kernel(
        _sc_merge_gather_body,
        out_type=jax.ShapeDtypeStruct((16, _D), jnp.float32),
        mesh=mesh,
        scratch_types=[
            pltpu.VMEM((_CPAD,), jnp.float32),
            pltpu.VMEM((_CPAD,), jnp.int32),
            pltpu.VMEM((16,), jnp.int32),
            pltpu.VMEM((16, _D), jnp.float32),
            pltpu.SemaphoreType.DMA,
        ],
        compiler_params=pltpu.CompilerParams(needs_layout_passes=False),
    )
    vflat = vals.reshape(-1)
    iflat = idx.reshape(-1)
    pad = _CPAD - _NCAND
    vflat = jnp.concatenate([vflat, jnp.full((pad,), _MINF, jnp.float32)])
    iflat = jnp.concatenate([iflat, jnp.zeros((pad,), jnp.int32)])
    return fn(vflat, iflat, memory)


def _sigmoid(x):
    return 1.0 / (1.0 + jnp.exp(-x))


def _softplus(x):
    m = jnp.maximum(x, 0.0)
    return m + jnp.log(jnp.exp(x - m) + jnp.exp(-m))


def _mamba_ln_kernel(rows_ref, inp_ref, cwt_ref, cb_ref, wdt_ref, wb_ref,
                     wc_ref, dtw_ref, dtb_ref, alt_ref, d_ref, opw_ref,
                     lnw_ref, lnb_ref, out_ref):
    dn = (((1,), (1,)), ((), ()))
    f32 = jnp.float32
    rows = rows_ref[0:_K, :]
    xz = lax.dot_general(rows, inp_ref[...], dn,
                         preferred_element_type=f32)
    x_in = xz[:, 0:_D_INNER]
    res = xz[:, _D_INNER:2 * _D_INNER]
    xpad = jnp.concatenate(
        [jnp.zeros((_D_CONV - 1, _D_INNER), f32), x_in], axis=0)
    conv = jnp.broadcast_to(cb_ref[...], (_K, _D_INNER))
    for j in range(_D_CONV):
        conv = conv + cwt_ref[j:j + 1, :] * xpad[j:j + _K, :]
    x_c = conv * _sigmoid(conv)

    dt = lax.dot_general(x_c, wdt_ref[...], dn,
                         preferred_element_type=f32)
    bt = lax.dot_general(wb_ref[...], x_c, dn,
                         preferred_element_type=f32)
    ct = lax.dot_general(wc_ref[...], x_c, dn,
                         preferred_element_type=f32)
    delta = _softplus(
        lax.dot_general(dt, dtw_ref[...], dn,
                        preferred_element_type=f32) + dtb_ref[...])
    a_t = -jnp.exp(alt_ref[...])

    h = jnp.zeros((_D_STATE, _D_INNER), f32)
    ys = []
    for t in range(_K):
        dt_t = delta[t:t + 1, :]
        xc_t = x_c[t:t + 1, :]
        b_t = bt[:, t:t + 1]
        c_t = ct[:, t:t + 1]
        h = jnp.exp(dt_t * a_t) * h + (dt_t * xc_t) * b_t
        ys.append(jnp.sum(h * c_t, axis=0, keepdims=True))
    y = jnp.concatenate(ys, axis=0)
    y = y + x_c * d_ref[...]
    y = y * (res * _sigmoid(res))
    out = lax.dot_general(y, opw_ref[...], dn,
                          preferred_element_type=f32)
    last = out[_K - 1:_K, :]
    mu = jnp.mean(last, axis=1, keepdims=True)
    var = jnp.mean((last - mu) ** 2, axis=1, keepdims=True)
    normed = (last - mu) / jnp.sqrt(var + 1e-5)
    out_ref[...] = normed * lnw_ref[...] + lnb_ref[...]


def _run_mamba_ln(rows, in_proj_w, conv_w, conv_b, x_proj_w, dt_proj_w,
                  dt_proj_b, A_log, D, out_proj_w, ln_w, ln_b):
    args = (
        rows,
        in_proj_w,
        conv_w.T,
        conv_b.reshape(1, _D_INNER),
        x_proj_w[:_DT_RANK],
        x_proj_w[_DT_RANK:_DT_RANK + _D_STATE],
        x_proj_w[_DT_RANK + _D_STATE:],
        dt_proj_w,
        dt_proj_b.reshape(1, _D_INNER),
        A_log.T,
        D.reshape(1, _D_INNER),
        out_proj_w,
        ln_w.reshape(1, _D),
        ln_b.reshape(1, _D),
    )
    return pl.pallas_call(
        _mamba_ln_kernel,
        out_shape=jax.ShapeDtypeStruct((1, _D), jnp.float32),
    )(*args)


def kernel(query, memory, in_proj_w, conv_w, conv_b, x_proj_w, dt_proj_w,
           dt_proj_b, A_log, D, out_proj_w, ln_w, ln_b):
    vals, idx = _run_sims_topk(query, memory)
    rows = _run_sc_merge_gather(vals, idx, memory)
    return _run_mamba_ln(rows, in_proj_w, conv_w, conv_b, x_proj_w,
                         dt_proj_w, dt_proj_b, A_log, D, out_proj_w,
                         ln_w, ln_b)


# speedup vs baseline: 5.5450x; 1.7993x over previous
import functools

import jax
import jax.numpy as jnp
from jax import lax
from jax.experimental import pallas as pl
from jax.experimental.pallas import tpu as pltpu
from jax.experimental.pallas import tpu_sc as plsc

_D = 128
_M = 1000000
_K = 8
_BLK = 16384
_NBLK = (_M + _BLK - 1) // _BLK
_RS = _BLK // 128
_NCAND = _NBLK * _K
_CHUNKS = (_NCAND + 15) // 16
_CPAD = _CHUNKS * 16
_MINF = float(jnp.finfo(jnp.float32).min)

_D_STATE = 16
_D_CONV = 4
_D_INNER = 256
_DT_RANK = 8


def _sims_topk_kernel(q_ref, mem_ref, vals_ref, idx_ref, th_ref):
    i = pl.program_id(0)

    @pl.when(i == 0)
    def _():
        th_ref[0] = _MINF

    q = q_ref[...]
    qn = q / jnp.maximum(jnp.sqrt(jnp.sum(q * q)), 1e-8)

    mem = mem_ref[...]
    dn = (((1,), (1,)), ((), ()))
    s = lax.dot_general(qn, mem, dn,
                        preferred_element_type=jnp.float32)
    ones = jnp.ones((1, _D), dtype=jnp.float32)
    n2 = lax.dot_general(ones, mem * mem, dn,
                         preferred_element_type=jnp.float32)
    sims = (s / jnp.maximum(jnp.sqrt(n2), 1e-8)).reshape(_RS, 128)

    gidx = (i * _BLK
            + lax.broadcasted_iota(jnp.int32, (_RS, 128), 0) * 128
            + lax.broadcasted_iota(jnp.int32, (_RS, 128), 1))
    sims = jnp.where(gidx < _M, sims, _MINF)

    bm = jnp.max(sims)
    th = th_ref[0]

    @pl.when(bm > th)
    def _extract():
        big = jnp.int32(2147483647)
        s2 = sims
        m = bm
        for j in range(_K):
            if j:
                m = jnp.max(s2)
            loc = jnp.min(jnp.where(s2 == m, gidx, big))
            vals_ref[0, 0, j] = m
            idx_ref[0, 0, j] = loc
            s2 = jnp.where(gidx == loc, _MINF, s2)
        th_ref[0] = jnp.maximum(th, m)

    @pl.when(bm <= th)
    def _skip():
        for j in range(_K):
            vals_ref[0, 0, j] = _MINF
            idx_ref[0, 0, j] = 0


def _run_sims_topk(query, memory):
    q2 = query.reshape(1, _D)
    return pl.pallas_call(
        _sims_topk_kernel,
        grid=(_NBLK,),
        in_specs=[
            pl.BlockSpec((1, _D), lambda i: (0, 0)),
            pl.BlockSpec((_BLK, _D), lambda i: (i, 0)),
        ],
        out_specs=[
            pl.BlockSpec((1, 1, _K), lambda i: (i, 0, 0),
                         memory_space=pltpu.SMEM),
            pl.BlockSpec((1, 1, _K), lambda i: (i, 0, 0),
                         memory_space=pltpu.SMEM),
        ],
        out_shape=[
            jax.ShapeDtypeStruct((_NBLK, 1, _K), jnp.float32),
            jax.ShapeDtypeStruct((_NBLK, 1, _K), jnp.int32),
        ],
        scratch_shapes=[pltpu.SMEM((1,), jnp.float32)],
    )(q2, memory)


def _sc_merge_gather_body(vals_hbm, idx_hbm, mem_hbm, out_hbm,
                          vals_v, idx_v, topi_v, rows_v, sem):
    cid = lax.axis_index("c")
    sid = lax.axis_index("s")

    @pl.when(jnp.logical_and(cid == 0, sid == 0))
    def _():
        pltpu.sync_copy(vals_hbm, vals_v)
        pltpu.sync_copy(idx_hbm, idx_v)

        def body(c, carry):
            cv, ci = carry
            v = vals_v[pl.ds(c * 16, 16)]
            ii = idx_v[pl.ds(c * 16, 16)]
            sv, si = plsc.sort_key_val(v, ii, descending=True)
            rv = lax.rev(sv, (0,))
            ri = lax.rev(si, (0,))
            keep = cv >= rv
            nv = jnp.where(keep, cv, rv)
            ni = jnp.where(keep, ci, ri)
            mv, mi = plsc.sort_key_val(nv, ni, descending=True)
            return (mv, mi)

        init = (jnp.full((16,), _MINF, jnp.float32),
                jnp.zeros((16,), jnp.int32))
        _, top_i = lax.fori_loop(0, _CHUNKS, body, init)
        topi_v[...] = top_i
        pltpu.async_copy(mem_hbm.at[topi_v], rows_v, sem).wait()
        pltpu.sync_copy(rows_v, out_hbm)


def _run_sc_merge_gather(vals, idx, memory):
    mesh = plsc.VectorSubcoreMesh(core_axis_name="c", subcore_axis_name="s",
                                  num_cores=2, num_subcores=16)
    fn = pl.kernel(
        _sc_merge_gather_body,
        out_type=jax.ShapeDtypeStruct((16, _D), jnp.float32),
        mesh=mesh,
        scratch_types=[
            pltpu.VMEM((_CPAD,), jnp.float32),
            pltpu.VMEM((_CPAD,), jnp.int32),
            pltpu.VMEM((16,), jnp.int32),
            pltpu.VMEM((16, _D), jnp.float32),
            pltpu.SemaphoreType.DMA,
        ],
        compiler_params=pltpu.CompilerParams(needs_layout_passes=False),
    )
    vflat = vals.reshape(-1)
    iflat = idx.reshape(-1)
    pad = _CPAD - _NCAND
    vflat = jnp.concatenate([vflat, jnp.full((pad,), _MINF, jnp.float32)])
    iflat = jnp.concatenate([iflat, jnp.zeros((pad,), jnp.int32)])
    return fn(vflat, iflat, memory)


def _sigmoid(x):
    return 1.0 / (1.0 + jnp.exp(-x))


def _softplus(x):
    m = jnp.maximum(x, 0.0)
    return m + jnp.log(jnp.exp(x - m) + jnp.exp(-m))


def _mamba_ln_kernel(rows_ref, inp_ref, cwt_ref, cb_ref, wdt_ref, wb_ref,
                     wc_ref, dtw_ref, dtb_ref, alt_ref, d_ref, opw_ref,
                     lnw_ref, lnb_ref, out_ref):
    dn = (((1,), (1,)), ((), ()))
    f32 = jnp.float32
    rows = rows_ref[0:_K, :]
    xz = lax.dot_general(rows, inp_ref[...], dn,
                         preferred_element_type=f32)
    x_in = xz[:, 0:_D_INNER]
    res = xz[:, _D_INNER:2 * _D_INNER]
    xpad = jnp.concatenate(
        [jnp.zeros((_D_CONV - 1, _D_INNER), f32), x_in], axis=0)
    conv = jnp.broadcast_to(cb_ref[...], (_K, _D_INNER))
    for j in range(_D_CONV):
        conv = conv + cwt_ref[j:j + 1, :] * xpad[j:j + _K, :]
    x_c = conv * _sigmoid(conv)

    dt = lax.dot_general(x_c, wdt_ref[...], dn,
                         preferred_element_type=f32)
    bt = lax.dot_general(wb_ref[...], x_c, dn,
                         preferred_element_type=f32)
    ct = lax.dot_general(wc_ref[...], x_c, dn,
                         preferred_element_type=f32)
    delta = _softplus(
        lax.dot_general(dt, dtw_ref[...], dn,
                        preferred_element_type=f32) + dtb_ref[...])
    a_t = -jnp.exp(alt_ref[...])

    h = jnp.zeros((_D_STATE, _D_INNER), f32)
    ys = []
    for t in range(_K):
        dt_t = delta[t:t + 1, :]
        xc_t = x_c[t:t + 1, :]
        b_t = bt[:, t:t + 1]
        c_t = ct[:, t:t + 1]
        h = jnp.exp(dt_t * a_t) * h + (dt_t * xc_t) * b_t
        ys.append(jnp.sum(h * c_t, axis=0, keepdims=True))
    y = jnp.concatenate(ys, axis=0)
    y = y + x_c * d_ref[...]
    y = y * (res * _sigmoid(res))
    out = lax.dot_general(y, opw_ref[...], dn,
                          preferred_element_type=f32)
    last = out[_K - 1:_K, :]
    mu = jnp.mean(last, axis=1, keepdims=True)
    var = jnp.mean((last - mu) ** 2, axis=1, keepdims=True)
    normed = (last - mu) / jnp.sqrt(var + 1e-5)
    out_ref[...] = normed * lnw_ref[...] + lnb_ref[...]


def _run_mamba_ln(rows, in_proj_w, conv_w, conv_b, x_proj_w, dt_proj_w,
                  dt_proj_b, A_log, D, out_proj_w, ln_w, ln_b):
    args = (
        rows,
        in_proj_w,
        conv_w.T,
        conv_b.reshape(1, _D_INNER),
        x_proj_w[:_DT_RANK],
        x_proj_w[_DT_RANK:_DT_RANK + _D_STATE],
        x_proj_w[_DT_RANK + _D_STATE:],
        dt_proj_w,
        dt_proj_b.reshape(1, _D_INNER),
        A_log.T,
        D.reshape(1, _D_INNER),
        out_proj_w,
        ln_w.reshape(1, _D),
        ln_b.reshape(1, _D),
    )
    return pl.pallas_call(
        _mamba_ln_kernel,
        out_shape=jax.ShapeDtypeStruct((1, _D), jnp.float32),
    )(*args)


def kernel(query, memory, in_proj_w, conv_w, conv_b, x_proj_w, dt_proj_w,
           dt_proj_b, A_log, D, out_proj_w, ln_w, ln_b):
    vals, idx = _run_sims_topk(query, memory)
    rows = _run_sc_merge_gather(vals, idx, memory)
    return _run_mamba_ln(rows, in_proj_w, conv_w, conv_b, x_proj_w,
                         dt_proj_w, dt_proj_b, A_log, D, out_proj_w,
                         ln_w, ln_b)


# speedup vs baseline: 7.2713x; 1.3113x over previous
import functools

import jax
import jax.numpy as jnp
from jax import lax
from jax.experimental import pallas as pl
from jax.experimental.pallas import tpu as pltpu
from jax.experimental.pallas import tpu_sc as plsc

_D = 128
_M = 1000000
_K = 8
_BLK = 16384
_NBLK = (_M + _BLK - 1) // _BLK
_RS = _BLK // 128
_NCAND = _NBLK * _K
_CHUNKS = (_NCAND + 15) // 16
_CPAD = _CHUNKS * 16
_MINF = float(jnp.finfo(jnp.float32).min)

_D_STATE = 16
_D_CONV = 4
_D_INNER = 256
_DT_RANK = 8


def _sims_topk_kernel(q_ref, mem_ref, vals_ref, idx_ref, r_ref):
    i = pl.program_id(0)

    @pl.when(i == 0)
    def _():
        for j in range(_K):
            r_ref[j] = _MINF

    q = q_ref[...]
    qn = q / jnp.maximum(jnp.sqrt(jnp.sum(q * q)), 1e-8)

    mem = mem_ref[...]
    dn = (((1,), (1,)), ((), ()))
    s = lax.dot_general(qn, mem, dn,
                        preferred_element_type=jnp.float32)
    ones = jnp.ones((1, _D), dtype=jnp.float32)
    n2 = lax.dot_general(ones, mem * mem, dn,
                         preferred_element_type=jnp.float32)
    sims = (s / jnp.maximum(jnp.sqrt(n2), 1e-8)).reshape(_RS, 128)

    gidx = (i * _BLK
            + lax.broadcasted_iota(jnp.int32, (_RS, 128), 0) * 128
            + lax.broadcasted_iota(jnp.int32, (_RS, 128), 1))
    sims = jnp.where(gidx < _M, sims, _MINF)

    bm = jnp.max(sims)
    th = r_ref[_K - 1]

    @pl.when(bm > th)
    def _extract():
        big = jnp.int32(2147483647)
        s2 = sims
        m = bm
        mv = []
        for j in range(_K):
            if j:
                m = jnp.max(s2)
            loc = jnp.min(jnp.where(s2 == m, gidx, big))
            vals_ref[0, 0, j] = m
            idx_ref[0, 0, j] = loc
            mv.append(m)
            s2 = jnp.where(gidx == loc, _MINF, s2)
        mm = [jnp.maximum(r_ref[j], mv[_K - 1 - j]) for j in range(_K)]
        for gap in (4, 2, 1):
            for j in range(_K):
                if (j % (2 * gap)) < gap:
                    a, b = mm[j], mm[j + gap]
                    mm[j] = jnp.maximum(a, b)
                    mm[j + gap] = jnp.minimum(a, b)
        for j in range(_K):
            r_ref[j] = mm[j]

    @pl.when(bm <= th)
    def _skip():
        for j in range(_K):
            vals_ref[0, 0, j] = _MINF
            idx_ref[0, 0, j] = 0


def _run_sims_topk(query, memory):
    q2 = query.reshape(1, _D)
    return pl.pallas_call(
        _sims_topk_kernel,
        grid=(_NBLK,),
        in_specs=[
            pl.BlockSpec((1, _D), lambda i: (0, 0)),
            pl.BlockSpec((_BLK, _D), lambda i: (i, 0)),
        ],
        out_specs=[
            pl.BlockSpec((1, 1, _K), lambda i: (i, 0, 0),
                         memory_space=pltpu.SMEM),
            pl.BlockSpec((1, 1, _K), lambda i: (i, 0, 0),
                         memory_space=pltpu.SMEM),
        ],
        out_shape=[
            jax.ShapeDtypeStruct((_NBLK, 1, _K), jnp.float32),
            jax.ShapeDtypeStruct((_NBLK, 1, _K), jnp.int32),
        ],
        scratch_shapes=[pltpu.SMEM((_K,), jnp.float32)],
    )(q2, memory)


def _sc_merge_gather_body(vals_hbm, idx_hbm, mem_hbm, out_hbm,
                          vals_v, idx_v, topi_v, rows_v, sem):
    cid = lax.axis_index("c")
    sid = lax.axis_index("s")

    @pl.when(jnp.logical_and(cid == 0, sid == 0))
    def _():
        pltpu.sync_copy(vals_hbm, vals_v)
        pltpu.sync_copy(idx_hbm, idx_v)

        def body(c, carry):
            cv, ci = carry
            v = vals_v[pl.ds(c * 16, 16)]
            ii = idx_v[pl.ds(c * 16, 16)]
            sv, si = plsc.sort_key_val(v, ii, descending=True)
            rv = lax.rev(sv, (0,))
            ri = lax.rev(si, (0,))
            keep = cv >= rv
            nv = jnp.where(keep, cv, rv)
            ni = jnp.where(keep, ci, ri)
            mv, mi = plsc.sort_key_val(nv, ni, descending=True)
            return (mv, mi)

        init = (jnp.full((16,), _MINF, jnp.float32),
                jnp.zeros((16,), jnp.int32))
        _, top_i = lax.fori_loop(0, _CHUNKS, body, init)
        topi_v[...] = top_i
        pltpu.async_copy(mem_hbm.at[topi_v], rows_v, sem).wait()
        pltpu.sync_copy(rows_v, out_hbm)


def _run_sc_merge_gather(vals, idx, memory):
    mesh = plsc.VectorSubcoreMesh(core_axis_name="c", subcore_axis_name="s",
                                  num_cores=2, num_subcores=16)
    fn = pl.kernel(
        _sc_merge_gather_body,
        out_type=jax.ShapeDtypeStruct((16, _D), jnp.float32),
        mesh=mesh,
        scratch_types=[
            pltpu.VMEM((_CPAD,), jnp.float32),
            pltpu.VMEM((_CPAD,), jnp.int32),
            pltpu.VMEM((16,), jnp.int32),
            pltpu.VMEM((16, _D), jnp.float32),
            pltpu.SemaphoreType.DMA,
        ],
        compiler_params=pltpu.CompilerParams(needs_layout_passes=False),
    )
    vflat = vals.reshape(-1)
    iflat = idx.reshape(-1)
    pad = _CPAD - _NCAND
    vflat = jnp.concatenate([vflat, jnp.full((pad,), _MINF, jnp.float32)])
    iflat = jnp.concatenate([iflat, jnp.zeros((pad,), jnp.int32)])
    return fn(vflat, iflat, memory)


def _sigmoid(x):
    return 1.0 / (1.0 + jnp.exp(-x))


def _softplus(x):
    m = jnp.maximum(x, 0.0)
    return m + jnp.log(jnp.exp(x - m) + jnp.exp(-m))


def _mamba_ln_kernel(rows_ref, inp_ref, cwt_ref, cb_ref, wdt_ref, wb_ref,
                     wc_ref, dtw_ref, dtb_ref, alt_ref, d_ref, opw_ref,
                     lnw_ref, lnb_ref, out_ref):
    dn = (((1,), (1,)), ((), ()))
    f32 = jnp.float32
    rows = rows_ref[0:_K, :]
    xz = lax.dot_general(rows, inp_ref[...], dn,
                         preferred_element_type=f32)
    x_in = xz[:, 0:_D_INNER]
    res = xz[:, _D_INNER:2 * _D_INNER]
    xpad = jnp.concatenate(
        [jnp.zeros((_D_CONV - 1, _D_INNER), f32), x_in], axis=0)
    conv = jnp.broadcast_to(cb_ref[...], (_K, _D_INNER))
    for j in range(_D_CONV):
        conv = conv + cwt_ref[j:j + 1, :] * xpad[j:j + _K, :]
    x_c = conv * _sigmoid(conv)

    dt = lax.dot_general(x_c, wdt_ref[...], dn,
                         preferred_element_type=f32)
    bt = lax.dot_general(wb_ref[...], x_c, dn,
                         preferred_element_type=f32)
    ct = lax.dot_general(wc_ref[...], x_c, dn,
                         preferred_element_type=f32)
    delta = _softplus(
        lax.dot_general(dt, dtw_ref[...], dn,
                        preferred_element_type=f32) + dtb_ref[...])
    a_t = -jnp.exp(alt_ref[...])

    h = jnp.zeros((_D_STATE, _D_INNER), f32)
    ys = []
    for t in range(_K):
        dt_t = delta[t:t + 1, :]
        xc_t = x_c[t:t + 1, :]
        b_t = bt[:, t:t + 1]
        c_t = ct[:, t:t + 1]
        h = jnp.exp(dt_t * a_t) * h + (dt_t * xc_t) * b_t
        ys.append(jnp.sum(h * c_t, axis=0, keepdims=True))
    y = jnp.concatenate(ys, axis=0)
    y = y + x_c * d_ref[...]
    y = y * (res * _sigmoid(res))
    out = lax.dot_general(y, opw_ref[...], dn,
                          preferred_element_type=f32)
    last = out[_K - 1:_K, :]
    mu = jnp.mean(last, axis=1, keepdims=True)
    var = jnp.mean((last - mu) ** 2, axis=1, keepdims=True)
    normed = (last - mu) / jnp.sqrt(var + 1e-5)
    out_ref[...] = normed * lnw_ref[...] + lnb_ref[...]


def _run_mamba_ln(rows, in_proj_w, conv_w, conv_b, x_proj_w, dt_proj_w,
                  dt_proj_b, A_log, D, out_proj_w, ln_w, ln_b):
    args = (
        rows,
        in_proj_w,
        conv_w.T,
        conv_b.reshape(1, _D_INNER),
        x_proj_w[:_DT_RANK],
        x_proj_w[_DT_RANK:_DT_RANK + _D_STATE],
        x_proj_w[_DT_RANK + _D_STATE:],
        dt_proj_w,
        dt_proj_b.reshape(1, _D_INNER),
        A_log.T,
        D.reshape(1, _D_INNER),
        out_proj_w,
        ln_w.reshape(1, _D),
        ln_b.reshape(1, _D),
    )
    return pl.pallas_call(
        _mamba_ln_kernel,
        out_shape=jax.ShapeDtypeStruct((1, _D), jnp.float32),
    )(*args)


def kernel(query, memory, in_proj_w, conv_w, conv_b, x_proj_w, dt_proj_w,
           dt_proj_b, A_log, D, out_proj_w, ln_w, ln_b):
    vals, idx = _run_sims_topk(query, memory)
    rows = _run_sc_merge_gather(vals, idx, memory)
    return _run_mamba_ln(rows, in_proj_w, conv_w, conv_b, x_proj_w,
                         dt_proj_w, dt_proj_b, A_log, D, out_proj_w,
                         ln_w, ln_b)


# speedup vs baseline: 7.8225x; 1.0758x over previous
import functools

import jax
import jax.numpy as jnp
from jax import lax
from jax.experimental import pallas as pl
from jax.experimental.pallas import tpu as pltpu
from jax.experimental.pallas import tpu_sc as plsc

_D = 128
_M = 1000000
_K = 8
_BLK = 16384
_NBLK = (_M + _BLK - 1) // _BLK
_RS = _BLK // 128
_NCAND = _NBLK * _K
_CHUNKS = (_NCAND + 15) // 16
_CPAD = _CHUNKS * 16
_MINF = float(jnp.finfo(jnp.float32).min)

_D_STATE = 16
_D_CONV = 4
_D_INNER = 256
_DT_RANK = 8


def _sims_topk_kernel(q_ref, mem_ref, vals_ref, idx_ref, r_ref):
    i = pl.program_id(0)

    @pl.when(i == 0)
    def _():
        for j in range(_K):
            r_ref[j] = _MINF

    q = q_ref[...]
    qn = q / jnp.maximum(jnp.sqrt(jnp.sum(q * q)), 1e-8)

    mem = mem_ref[...]
    dn = (((1,), (1,)), ((), ()))
    s = lax.dot_general(qn, mem, dn,
                        preferred_element_type=jnp.float32)
    ones = jnp.ones((1, _D), dtype=jnp.float32)
    n2 = lax.dot_general(ones, mem * mem, dn,
                         preferred_element_type=jnp.float32)
    sims = (s / jnp.maximum(jnp.sqrt(n2), 1e-8)).reshape(_RS, 128)

    gidx = (i * _BLK
            + lax.broadcasted_iota(jnp.int32, (_RS, 128), 0) * 128
            + lax.broadcasted_iota(jnp.int32, (_RS, 128), 1))
    sims = jnp.where(gidx < _M, sims, _MINF)

    bm = jnp.max(sims)
    th = r_ref[_K - 1]

    @pl.when(bm > th)
    def _extract():
        big = jnp.int32(2147483647)
        cnt = jnp.sum(jnp.where(sims > th, 1, 0).astype(jnp.int32))

        def _run(niter):
            def _go():
                s2 = sims
                m = bm
                mv = []
                for j in range(_K):
                    if j < niter:
                        if j:
                            m = jnp.max(s2)
                        loc = jnp.min(jnp.where(s2 == m, gidx, big))
                        vals_ref[0, 0, j] = m
                        idx_ref[0, 0, j] = loc
                        mv.append(m)
                        s2 = jnp.where(gidx == loc, _MINF, s2)
                    else:
                        vals_ref[0, 0, j] = _MINF
                        idx_ref[0, 0, j] = 0
                        mv.append(jnp.float32(_MINF))
                mm = [jnp.maximum(r_ref[j], mv[_K - 1 - j])
                      for j in range(_K)]
                for gap in (4, 2, 1):
                    for j in range(_K):
                        if (j % (2 * gap)) < gap:
                            a, b = mm[j], mm[j + gap]
                            mm[j] = jnp.maximum(a, b)
                            mm[j + gap] = jnp.minimum(a, b)
                for j in range(_K):
                    r_ref[j] = mm[j]
            return _go

        pl.when(cnt <= 3)(_run(3))
        pl.when(cnt > 3)(_run(_K))

    @pl.when(bm <= th)
    def _skip():
        for j in range(_K):
            vals_ref[0, 0, j] = _MINF
            idx_ref[0, 0, j] = 0


def _run_sims_topk(query, memory):
    q2 = query.reshape(1, _D)
    return pl.pallas_call(
        _sims_topk_kernel,
        grid=(_NBLK,),
        in_specs=[
            pl.BlockSpec((1, _D), lambda i: (0, 0)),
            pl.BlockSpec((_BLK, _D), lambda i: (i, 0)),
        ],
        out_specs=[
            pl.BlockSpec((1, 1, _K), lambda i: (i, 0, 0),
                         memory_space=pltpu.SMEM),
            pl.BlockSpec((1, 1, _K), lambda i: (i, 0, 0),
                         memory_space=pltpu.SMEM),
        ],
        out_shape=[
            jax.ShapeDtypeStruct((_NBLK, 1, _K), jnp.float32),
            jax.ShapeDtypeStruct((_NBLK, 1, _K), jnp.int32),
        ],
        scratch_shapes=[pltpu.SMEM((_K,), jnp.float32)],
    )(q2, memory)


def _sc_merge_gather_body(vals_hbm, idx_hbm, mem_hbm, out_hbm,
                          vals_v, idx_v, topi_v, rows_v, sem):
    cid = lax.axis_index("c")
    sid = lax.axis_index("s")

    @pl.when(jnp.logical_and(cid == 0, sid == 0))
    def _():
        pltpu.sync_copy(vals_hbm, vals_v)
        pltpu.sync_copy(idx_hbm, idx_v)

        def body(c, carry):
            cv, ci = carry
            v = vals_v[pl.ds(c * 16, 16)]
            ii = idx_v[pl.ds(c * 16, 16)]
            sv, si = plsc.sort_key_val(v, ii, descending=True)
            rv = lax.rev(sv, (0,))
            ri = lax.rev(si, (0,))
            keep = cv >= rv
            nv = jnp.where(keep, cv, rv)
            ni = jnp.where(keep, ci, ri)
            mv, mi = plsc.sort_key_val(nv, ni, descending=True)
            return (mv, mi)

        init = (jnp.full((16,), _MINF, jnp.float32),
                jnp.zeros((16,), jnp.int32))
        _, top_i = lax.fori_loop(0, _CHUNKS, body, init)
        topi_v[...] = top_i
        pltpu.async_copy(mem_hbm.at[topi_v], rows_v, sem).wait()
        pltpu.sync_copy(rows_v, out_hbm)


def _run_sc_merge_gather(vals, idx, memory):
    mesh = plsc.VectorSubcoreMesh(core_axis_name="c", subcore_axis_name="s",
                                  num_cores=2, num_subcores=16)
    fn = pl.kernel(
        _sc_merge_gather_body,
        out_type=jax.ShapeDtypeStruct((16, _D), jnp.float32),
        mesh=mesh,
        scratch_types=[
            pltpu.VMEM((_CPAD,), jnp.float32),
            pltpu.VMEM((_CPAD,), jnp.int32),
            pltpu.VMEM((16,), jnp.int32),
            pltpu.VMEM((16, _D), jnp.float32),
            pltpu.SemaphoreType.DMA,
        ],
        compiler_params=pltpu.CompilerParams(needs_layout_passes=False),
    )
    vflat = vals.reshape(-1)
    iflat = idx.reshape(-1)
    pad = _CPAD - _NCAND
    vflat = jnp.concatenate([vflat, jnp.full((pad,), _MINF, jnp.float32)])
    iflat = jnp.concatenate([iflat, jnp.zeros((pad,), jnp.int32)])
    return fn(vflat, iflat, memory)


def _sigmoid(x):
    return 1.0 / (1.0 + jnp.exp(-x))


def _softplus(x):
    m = jnp.maximum(x, 0.0)
    return m + jnp.log(jnp.exp(x - m) + jnp.exp(-m))


def _mamba_ln_kernel(rows_ref, inp_ref, cwt_ref, cb_ref, wdt_ref, wb_ref,
                     wc_ref, dtw_ref, dtb_ref, alt_ref, d_ref, opw_ref,
                     lnw_ref, lnb_ref, out_ref):
    dn = (((1,), (1,)), ((), ()))
    f32 = jnp.float32
    rows = rows_ref[0:_K, :]
    xz = lax.dot_general(rows, inp_ref[...], dn,
                         preferred_element_type=f32)
    x_in = xz[:, 0:_D_INNER]
    res = xz[:, _D_INNER:2 * _D_INNER]
    xpad = jnp.concatenate(
        [jnp.zeros((_D_CONV - 1, _D_INNER), f32), x_in], axis=0)
    conv = jnp.broadcast_to(cb_ref[...], (_K, _D_INNER))
    for j in range(_D_CONV):
        conv = conv + cwt_ref[j:j + 1, :] * xpad[j:j + _K, :]
    x_c = conv * _sigmoid(conv)

    dt = lax.dot_general(x_c, wdt_ref[...], dn,
                         preferred_element_type=f32)
    bt = lax.dot_general(wb_ref[...], x_c, dn,
                         preferred_element_type=f32)
    ct = lax.dot_general(wc_ref[...], x_c, dn,
                         preferred_element_type=f32)
    delta = _softplus(
        lax.dot_general(dt, dtw_ref[...], dn,
                        preferred_element_type=f32) + dtb_ref[...])
    a_t = -jnp.exp(alt_ref[...])

    h = jnp.zeros((_D_STATE, _D_INNER), f32)
    ys = []
    for t in range(_K):
        dt_t = delta[t:t + 1, :]
        xc_t = x_c[t:t + 1, :]
        b_t = bt[:, t:t + 1]
        c_t = ct[:, t:t + 1]
        h = jnp.exp(dt_t * a_t) * h + (dt_t * xc_t) * b_t
        ys.append(jnp.sum(h * c_t, axis=0, keepdims=True))
    y = jnp.concatenate(ys, axis=0)
    y = y + x_c * d_ref[...]
    y = y * (res * _sigmoid(res))
    out = lax.dot_general(y, opw_ref[...], dn,
                          preferred_element_type=f32)
    last = out[_K - 1:_K, :]
    mu = jnp.mean(last, axis=1, keepdims=True)
    var = jnp.mean((last - mu) ** 2, axis=1, keepdims=True)
    normed = (last - mu) / jnp.sqrt(var + 1e-5)
    out_ref[...] = normed * lnw_ref[...] + lnb_ref[...]


def _run_mamba_ln(rows, in_proj_w, conv_w, conv_b, x_proj_w, dt_proj_w,
                  dt_proj_b, A_log, D, out_proj_w, ln_w, ln_b):
    args = (
        rows,
        in_proj_w,
        conv_w.T,
        conv_b.reshape(1, _D_INNER),
        x_proj_w[:_DT_RANK],
        x_proj_w[_DT_RANK:_DT_RANK + _D_STATE],
        x_proj_w[_DT_RANK + _D_STATE:],
        dt_proj_w,
        dt_proj_b.reshape(1, _D_INNER),
        A_log.T,
        D.reshape(1, _D_INNER),
        out_proj_w,
        ln_w.reshape(1, _D),
        ln_b.reshape(1, _D),
    )
    return pl.pallas_call(
        _mamba_ln_kernel,
        out_shape=jax.ShapeDtypeStruct((1, _D), jnp.float32),
    )(*args)


def kernel(query, memory, in_proj_w, conv_w, conv_b, x_proj_w, dt_proj_w,
           dt_proj_b, A_log, D, out_proj_w, ln_w, ln_b):
    vals, idx = _run_sims_topk(query, memory)
    rows = _run_sc_merge_gather(vals, idx, memory)
    return _run_mamba_ln(rows, in_proj_w, conv_w, conv_b, x_proj_w,
                         dt_proj_w, dt_proj_b, A_log, D, out_proj_w,
                         ln_w, ln_b)


# speedup vs baseline: 8.9317x; 1.1418x over previous
import functools

import jax
import jax.numpy as jnp
from jax import lax
from jax.experimental import pallas as pl
from jax.experimental.pallas import tpu as pltpu
from jax.experimental.pallas import tpu_sc as plsc

_D = 128
_M = 1000000
_K = 8
_BLK = 32768
_NBLK = (_M + _BLK - 1) // _BLK
_RS = _BLK // 128
_NCAND = _NBLK * _K
_CHUNKS = (_NCAND + 15) // 16
_CPAD = _CHUNKS * 16
_MINF = float(jnp.finfo(jnp.float32).min)

_D_STATE = 16
_D_CONV = 4
_D_INNER = 256
_DT_RANK = 8


def _sims_topk_kernel(q_ref, mem_ref, vals_ref, idx_ref, r_ref):
    i = pl.program_id(0)

    @pl.when(i == 0)
    def _():
        for j in range(_K):
            r_ref[j] = _MINF

    q = q_ref[...]
    qn = q / jnp.maximum(jnp.sqrt(jnp.sum(q * q)), 1e-8)

    mem = mem_ref[...]
    dn = (((1,), (1,)), ((), ()))
    s = lax.dot_general(qn, mem, dn,
                        preferred_element_type=jnp.float32)
    ones = jnp.ones((1, _D), dtype=jnp.float32)
    n2 = lax.dot_general(ones, mem * mem, dn,
                         preferred_element_type=jnp.float32)
    sims = (s / jnp.maximum(jnp.sqrt(n2), 1e-8)).reshape(_RS, 128)

    gidx = (i * _BLK
            + lax.broadcasted_iota(jnp.int32, (_RS, 128), 0) * 128
            + lax.broadcasted_iota(jnp.int32, (_RS, 128), 1))
    sims = jnp.where(gidx < _M, sims, _MINF)

    bm = jnp.max(sims)
    th = r_ref[_K - 1]

    @pl.when(bm > th)
    def _extract():
        big = jnp.int32(2147483647)
        cnt = jnp.sum(jnp.where(sims > th, 1, 0).astype(jnp.int32))

        def _run(niter):
            def _go():
                s2 = sims
                m = bm
                mv = []
                for j in range(_K):
                    if j < niter:
                        if j:
                            m = jnp.max(s2)
                        loc = jnp.min(jnp.where(s2 == m, gidx, big))
                        vals_ref[0, 0, j] = m
                        idx_ref[0, 0, j] = loc
                        mv.append(m)
                        s2 = jnp.where(gidx == loc, _MINF, s2)
                    else:
                        vals_ref[0, 0, j] = _MINF
                        idx_ref[0, 0, j] = 0
                        mv.append(jnp.float32(_MINF))
                mm = [jnp.maximum(r_ref[j], mv[_K - 1 - j])
                      for j in range(_K)]
                for gap in (4, 2, 1):
                    for j in range(_K):
                        if (j % (2 * gap)) < gap:
                            a, b = mm[j], mm[j + gap]
                            mm[j] = jnp.maximum(a, b)
                            mm[j + gap] = jnp.minimum(a, b)
                for j in range(_K):
                    r_ref[j] = mm[j]
            return _go

        pl.when(cnt <= 3)(_run(3))
        pl.when(cnt > 3)(_run(_K))

    @pl.when(bm <= th)
    def _skip():
        for j in range(_K):
            vals_ref[0, 0, j] = _MINF
            idx_ref[0, 0, j] = 0


def _run_sims_topk(query, memory):
    q2 = query.reshape(1, _D)
    return pl.pallas_call(
        _sims_topk_kernel,
        grid=(_NBLK,),
        in_specs=[
            pl.BlockSpec((1, _D), lambda i: (0, 0)),
            pl.BlockSpec((_BLK, _D), lambda i: (i, 0)),
        ],
        out_specs=[
            pl.BlockSpec((1, 1, _K), lambda i: (i, 0, 0),
                         memory_space=pltpu.SMEM),
            pl.BlockSpec((1, 1, _K), lambda i: (i, 0, 0),
                         memory_space=pltpu.SMEM),
        ],
        out_shape=[
            jax.ShapeDtypeStruct((_NBLK, 1, _K), jnp.float32),
            jax.ShapeDtypeStruct((_NBLK, 1, _K), jnp.int32),
        ],
        scratch_shapes=[pltpu.SMEM((_K,), jnp.float32)],
    )(q2, memory)


def _sc_merge_gather_body(vals_hbm, idx_hbm, mem_hbm, out_hbm,
                          vals_v, idx_v, topi_v, rows_v, sem):
    cid = lax.axis_index("c")
    sid = lax.axis_index("s")

    @pl.when(jnp.logical_and(cid == 0, sid == 0))
    def _():
        pltpu.sync_copy(vals_hbm, vals_v)
        pltpu.sync_copy(idx_hbm, idx_v)

        def body(c, carry):
            cv, ci = carry
            v = vals_v[pl.ds(c * 16, 16)]
            ii = idx_v[pl.ds(c * 16, 16)]
            sv, si = plsc.sort_key_val(v, ii, descending=True)
            rv = lax.rev(sv, (0,))
            ri = lax.rev(si, (0,))
            keep = cv >= rv
            nv = jnp.where(keep, cv, rv)
            ni = jnp.where(keep, ci, ri)
            mv, mi = plsc.sort_key_val(nv, ni, descending=True)
            return (mv, mi)

        init = (jnp.full((16,), _MINF, jnp.float32),
                jnp.zeros((16,), jnp.int32))
        _, top_i = lax.fori_loop(0, _CHUNKS, body, init)
        topi_v[...] = top_i
        pltpu.async_copy(mem_hbm.at[topi_v], rows_v, sem).wait()
        pltpu.sync_copy(rows_v, out_hbm)


def _run_sc_merge_gather(vals, idx, memory):
    mesh = plsc.VectorSubcoreMesh(core_axis_name="c", subcore_axis_name="s",
                                  num_cores=2, num_subcores=16)
    fn = pl.kernel(
        _sc_merge_gather_body,
        out_type=jax.ShapeDtypeStruct((16, _D), jnp.float32),
        mesh=mesh,
        scratch_types=[
            pltpu.VMEM((_CPAD,), jnp.float32),
            pltpu.VMEM((_CPAD,), jnp.int32),
            pltpu.VMEM((16,), jnp.int32),
            pltpu.VMEM((16, _D), jnp.float32),
            pltpu.SemaphoreType.DMA,
        ],
        compiler_params=pltpu.CompilerParams(needs_layout_passes=False),
    )
    vflat = vals.reshape(-1)
    iflat = idx.reshape(-1)
    pad = _CPAD - _NCAND
    vflat = jnp.concatenate([vflat, jnp.full((pad,), _MINF, jnp.float32)])
    iflat = jnp.concatenate([iflat, jnp.zeros((pad,), jnp.int32)])
    return fn(vflat, iflat, memory)


def _sigmoid(x):
    return 1.0 / (1.0 + jnp.exp(-x))


def _softplus(x):
    m = jnp.maximum(x, 0.0)
    return m + jnp.log(jnp.exp(x - m) + jnp.exp(-m))


def _mamba_ln_kernel(rows_ref, inp_ref, cwt_ref, cb_ref, wdt_ref, wb_ref,
                     wc_ref, dtw_ref, dtb_ref, alt_ref, d_ref, opw_ref,
                     lnw_ref, lnb_ref, out_ref):
    dn = (((1,), (1,)), ((), ()))
    f32 = jnp.float32
    rows = rows_ref[0:_K, :]
    xz = lax.dot_general(rows, inp_ref[...], dn,
                         preferred_element_type=f32)
    x_in = xz[:, 0:_D_INNER]
    res = xz[:, _D_INNER:2 * _D_INNER]
    xpad = jnp.concatenate(
        [jnp.zeros((_D_CONV - 1, _D_INNER), f32), x_in], axis=0)
    conv = jnp.broadcast_to(cb_ref[...], (_K, _D_INNER))
    for j in range(_D_CONV):
        conv = conv + cwt_ref[j:j + 1, :] * xpad[j:j + _K, :]
    x_c = conv * _sigmoid(conv)

    dt = lax.dot_general(x_c, wdt_ref[...], dn,
                         preferred_element_type=f32)
    bt = lax.dot_general(wb_ref[...], x_c, dn,
                         preferred_element_type=f32)
    ct = lax.dot_general(wc_ref[...], x_c, dn,
                         preferred_element_type=f32)
    delta = _softplus(
        lax.dot_general(dt, dtw_ref[...], dn,
                        preferred_element_type=f32) + dtb_ref[...])
    a_t = -jnp.exp(alt_ref[...])

    h = jnp.zeros((_D_STATE, _D_INNER), f32)
    ys = []
    for t in range(_K):
        dt_t = delta[t:t + 1, :]
        xc_t = x_c[t:t + 1, :]
        b_t = bt[:, t:t + 1]
        c_t = ct[:, t:t + 1]
        h = jnp.exp(dt_t * a_t) * h + (dt_t * xc_t) * b_t
        ys.append(jnp.sum(h * c_t, axis=0, keepdims=True))
    y = jnp.concatenate(ys, axis=0)
    y = y + x_c * d_ref[...]
    y = y * (res * _sigmoid(res))
    out = lax.dot_general(y, opw_ref[...], dn,
                          preferred_element_type=f32)
    last = out[_K - 1:_K, :]
    mu = jnp.mean(last, axis=1, keepdims=True)
    var = jnp.mean((last - mu) ** 2, axis=1, keepdims=True)
    normed = (last - mu) / jnp.sqrt(var + 1e-5)
    out_ref[...] = normed * lnw_ref[...] + lnb_ref[...]


def _run_mamba_ln(rows, in_proj_w, conv_w, conv_b, x_proj_w, dt_proj_w,
                  dt_proj_b, A_log, D, out_proj_w, ln_w, ln_b):
    args = (
        rows,
        in_proj_w,
        conv_w.T,
        conv_b.reshape(1, _D_INNER),
        x_proj_w[:_DT_RANK],
        x_proj_w[_DT_RANK:_DT_RANK + _D_STATE],
        x_proj_w[_DT_RANK + _D_STATE:],
        dt_proj_w,
        dt_proj_b.reshape(1, _D_INNER),
        A_log.T,
        D.reshape(1, _D_INNER),
        out_proj_w,
        ln_w.reshape(1, _D),
        ln_b.reshape(1, _D),
    )
    return pl.pallas_call(
        _mamba_ln_kernel,
        out_shape=jax.ShapeDtypeStruct((1, _D), jnp.float32),
    )(*args)


def kernel(query, memory, in_proj_w, conv_w, conv_b, x_proj_w, dt_proj_w,
           dt_proj_b, A_log, D, out_proj_w, ln_w, ln_b):
    vals, idx = _run_sims_topk(query, memory)
    rows = _run_sc_merge_gather(vals, idx, memory)
    return _run_mamba_ln(rows, in_proj_w, conv_w, conv_b, x_proj_w,
                         dt_proj_w, dt_proj_b, A_log, D, out_proj_w,
                         ln_w, ln_b)
